# baseline (speedup 1.0000x reference)
import functools

import jax
import jax.numpy as jnp
from jax.experimental import pallas as pl
from jax.experimental.pallas import tpu as pltpu

_NSPLIT = 4


def _moe_kernel(nsplit, x_ref, wg_ref, bg_ref, be_ref, *rest):
    we_refs = rest[:nsplit]
    out_ref = rest[nsplit]
    s_ref, cw_ref = rest[nsplit + 1:]
    e = pl.program_id(0)

    @pl.when(e == 0)
    def _():
        x = x_ref[...]
        logits = jax.lax.dot_general(
            x, wg_ref[...], (((1,), (1,)), ((), ())),
            preferred_element_type=jnp.float32) + bg_ref[...]
        E = logits.shape[1]
        eids = jax.lax.broadcasted_iota(jnp.int32, logits.shape, 1)
        v1 = jnp.max(logits, axis=1, keepdims=True)
        i1 = jnp.min(jnp.where(logits == v1, eids, E), axis=1, keepdims=True)
        oh1 = eids == i1
        masked = jnp.where(oh1, -jnp.inf, logits)
        v2 = jnp.max(masked, axis=1, keepdims=True)
        i2 = jnp.min(jnp.where(masked == v2, eids, E), axis=1, keepdims=True)
        oh2 = eids == i2
        t = jnp.exp(v2 - v1)
        w1 = 1.0 / (1.0 + t)
        w2 = t / (1.0 + t)
        coef = w1 * oh1.astype(jnp.float32) + w2 * oh2.astype(jnp.float32)
        s_ref[...] = jax.lax.dot_general(
            coef, x, (((0,), (0,)), ((), ())),
            preferred_element_type=jnp.float32)
        cw_ref[...] = jnp.sum(coef, axis=0, keepdims=True)

    contrib = jax.lax.dot_general(
        s_ref[pl.ds(e * nsplit, 1), :], we_refs[0][0],
        (((1,), (1,)), ((), ())),
        preferred_element_type=jnp.float32)
    for j in range(1, nsplit):
        contrib = contrib + jax.lax.dot_general(
            s_ref[pl.ds(e * nsplit + j, 1), :], we_refs[j][0],
            (((1,), (1,)), ((), ())),
            preferred_element_type=jnp.float32)

    @pl.when(e == 0)
    def _():
        bias = jax.lax.dot_general(
            cw_ref[...], be_ref[...], (((1,), (0,)), ((), ())),
            preferred_element_type=jnp.float32)
        out_ref[...] = contrib + bias

    @pl.when(e != 0)
    def _():
        out_ref[...] = out_ref[...] + contrib


def kernel(x, Wg, bg, We, be):
    B, D = x.shape
    E, O, _ = We.shape
    ns = _NSPLIT
    we_specs = [
        pl.BlockSpec((1, O, D), functools.partial(
            lambda e, j: (e * ns + j, 0, 0), j=j))
        for j in range(ns)
    ]
    total = pl.pallas_call(
        functools.partial(_moe_kernel, ns),
        grid=(E // ns,),
        in_specs=[
            pl.BlockSpec((B, D), lambda e: (0, 0)),
            pl.BlockSpec((E, D), lambda e: (0, 0)),
            pl.BlockSpec((1, E), lambda e: (0, 0)),
            pl.BlockSpec((E, O), lambda e: (0, 0)),
        ] + we_specs,
        out_specs=pl.BlockSpec((1, O), lambda e: (0, 0)),
        out_shape=jax.ShapeDtypeStruct((1, O), jnp.float32),
        scratch_shapes=[
            pltpu.VMEM((E, D), jnp.float32),
            pltpu.VMEM((1, E), jnp.float32),
        ],
    )(x, Wg, bg.reshape(1, E), be, *([We] * ns))
    return jnp.broadcast_to(total, (B, O)).astype(x.dtype)


# speedup vs baseline: 1.0982x; 1.0982x over previous
import functools

import jax
import jax.numpy as jnp
from jax.experimental import pallas as pl
from jax.experimental.pallas import tpu as pltpu

_NSPLIT = 2


def _moe_kernel(nsplit, x_ref, wg_ref, bg_ref, be_ref, *rest):
    we_refs = rest[:nsplit]
    out_ref = rest[nsplit]
    s_ref, cw_ref, tot_ref = rest[nsplit + 1:]
    e = pl.program_id(0)

    @pl.when(e == 0)
    def _():
        x = x_ref[...]
        logits = jax.lax.dot_general(
            x, wg_ref[...], (((1,), (1,)), ((), ())),
            preferred_element_type=jnp.float32) + bg_ref[...]
        E = logits.shape[1]
        eids = jax.lax.broadcasted_iota(jnp.int32, logits.shape, 1)
        v1 = jnp.max(logits, axis=1, keepdims=True)
        i1 = jnp.min(jnp.where(logits == v1, eids, E), axis=1, keepdims=True)
        oh1 = eids == i1
        masked = jnp.where(oh1, -jnp.inf, logits)
        v2 = jnp.max(masked, axis=1, keepdims=True)
        i2 = jnp.min(jnp.where(masked == v2, eids, E), axis=1, keepdims=True)
        oh2 = eids == i2
        t = jnp.exp(v2 - v1)
        w1 = 1.0 / (1.0 + t)
        w2 = t / (1.0 + t)
        coef = w1 * oh1.astype(jnp.float32) + w2 * oh2.astype(jnp.float32)
        s_ref[...] = jax.lax.dot_general(
            coef, x, (((0,), (0,)), ((), ())),
            preferred_element_type=jnp.float32)
        cw_ref[...] = jnp.sum(coef, axis=0, keepdims=True)

    contrib = jax.lax.dot_general(
        s_ref[pl.ds(e * nsplit, 1), :], we_refs[0][0],
        (((1,), (1,)), ((), ())),
        preferred_element_type=jnp.float32)
    for j in range(1, nsplit):
        contrib = contrib + jax.lax.dot_general(
            s_ref[pl.ds(e * nsplit + j, 1), :], we_refs[j][0],
            (((1,), (1,)), ((), ())),
            preferred_element_type=jnp.float32)

    @pl.when(e == 0)
    def _():
        bias = jax.lax.dot_general(
            cw_ref[...], be_ref[...], (((1,), (0,)), ((), ())),
            preferred_element_type=jnp.float32)
        tot_ref[...] = contrib + bias

    @pl.when(e != 0)
    def _():
        tot_ref[...] = tot_ref[...] + contrib

    @pl.when(e == pl.num_programs(0) - 1)
    def _():
        out_ref[...] = jnp.broadcast_to(tot_ref[...], out_ref.shape)


def kernel(x, Wg, bg, We, be):
    B, D = x.shape
    E, O, _ = We.shape
    ns = _NSPLIT
    we_specs = [
        pl.BlockSpec((1, O, D), functools.partial(
            lambda e, j: (e * ns + j, 0, 0), j=j))
        for j in range(ns)
    ]
    total = pl.pallas_call(
        functools.partial(_moe_kernel, ns),
        grid=(E // ns,),
        in_specs=[
            pl.BlockSpec((B, D), lambda e: (0, 0)),
            pl.BlockSpec((E, D), lambda e: (0, 0)),
            pl.BlockSpec((1, E), lambda e: (0, 0)),
            pl.BlockSpec((E, O), lambda e: (0, 0)),
        ] + we_specs,
        out_specs=pl.BlockSpec((B, O), lambda e: (0, 0)),
        out_shape=jax.ShapeDtypeStruct((B, O), jnp.float32),
        scratch_shapes=[
            pltpu.VMEM((E, D), jnp.float32),
            pltpu.VMEM((1, E), jnp.float32),
            pltpu.VMEM((1, O), jnp.float32),
        ],
    )(x, Wg, bg.reshape(1, E), be, *([We] * ns))
    return total.astype(x.dtype)
